# baseline (speedup 1.0000x reference)
import functools

import jax
import jax.numpy as jnp
from jax import lax
from jax.experimental import pallas as pl
from jax.experimental.pallas import tpu as pltpu
from jax.experimental.pallas import tpu_sc as plsc

N = 10000
D = 256
DE = 16
E = 160000

NC = 2
NS = 16
NW = NC * NS
EPW = 5120
E_PAD = NW * EPW
CH = 128
NCH = EPW // CH
N_A = 10240
RPT = N_A // NS
TW = 2 * N_A
UNROLL = CH // 16


def _proj_body(wc_ref, bc_ref, x_ref, we_ref, bm_ref, ea_ref, p_ref, el_ref):
    @pl.when(pl.program_id(0) == 0)
    def _():
        p_ref[...] = lax.dot_general(
            wc_ref[...], x_ref[...], (((0,), (1,)), ((), ())),
            preferred_element_type=jnp.float32) + bc_ref[...]

    el_ref[...] = jnp.dot(we_ref[...], ea_ref[...],
                          preferred_element_type=jnp.float32) + bm_ref[...]


def _proj(wc, bc_t, x, we_t, bm_t, ea_t):
    return pl.pallas_call(
        _proj_body,
        grid=(10,),
        in_specs=[
            pl.BlockSpec((D, 4), lambda i: (0, 0)),
            pl.BlockSpec((4, 1), lambda i: (0, 0)),
            pl.BlockSpec((N, D), lambda i: (0, 0)),
            pl.BlockSpec((2, DE), lambda i: (0, 0)),
            pl.BlockSpec((2, 1), lambda i: (0, 0)),
            pl.BlockSpec((DE, E_PAD // 10), lambda i: (0, i)),
        ],
        out_specs=[
            pl.BlockSpec((4, N), lambda i: (0, 0)),
            pl.BlockSpec((2, E_PAD // 10), lambda i: (0, i)),
        ],
        out_shape=[
            jax.ShapeDtypeStruct((4, N), jnp.float32),
            jax.ShapeDtypeStruct((2, E_PAD), jnp.float32),
        ],
    )(wc, bc_t, x, we_t, bm_t, ea_t)


def _finish_body(p4_ref, p_ref, o_ref):
    p = p_ref[...]
    z0 = p4_ref[pl.ds(0, N)] + p[0, 0, :N] + p[1, 0, :N]
    z1 = p4_ref[pl.ds(N, N)] + p[0, 1, :N] + p[1, 1, :N]
    m = jnp.maximum(z0, z1)
    l = m + jnp.log(jnp.exp(z0 - m) + jnp.exp(z1 - m))
    o_ref[0, :] = z0 - l
    o_ref[1, :] = z1 - l


def _finish(proj_flat, parts):
    return pl.pallas_call(
        _finish_body,
        out_shape=jax.ShapeDtypeStruct((2, N), jnp.float32),
    )(proj_flat, parts)


def _sc_body(proj_hbm, ei_hbm, el_hbm, z_hbm, out_hbm,
             table_v, src_v, dst_v, vals0_v, vals1_v,
             agg0_sh, agg1_sh, sem_in, sem_s):
    cid = lax.axis_index("c")
    sid = lax.axis_index("s")
    wid = cid * NS + sid

    cps = [
        pltpu.async_copy(z_hbm.at[pl.ds(sid * RPT, RPT)],
                         agg0_sh.at[pl.ds(sid * RPT, RPT)], sem_in),
        pltpu.async_copy(z_hbm.at[pl.ds(sid * RPT, RPT)],
                         agg1_sh.at[pl.ds(sid * RPT, RPT)], sem_in),
        pltpu.async_copy(proj_hbm.at[pl.ds(2 * N, N)],
                         table_v.at[pl.ds(0, N)], sem_in),
        pltpu.async_copy(proj_hbm.at[pl.ds(3 * N, N)],
                         table_v.at[pl.ds(N_A, N)], sem_in),
        pltpu.async_copy(ei_hbm.at[0, wid], src_v, sem_in),
        pltpu.async_copy(ei_hbm.at[1, wid], dst_v, sem_in),
        pltpu.async_copy(el_hbm.at[0, pl.ds(wid * EPW, EPW)], vals0_v, sem_in),
        pltpu.async_copy(el_hbm.at[1, pl.ds(wid * EPW, EPW)], vals1_v, sem_in),
    ]
    for c in cps:
        c.wait()
    plsc.subcore_barrier()

    def chunk(j, carry):
        for k in range(UNROLL):
            s16 = src_v[j, pl.ds(k * 16, 16)]
            g0 = plsc.load_gather(table_v, [s16])
            g1 = plsc.load_gather(table_v, [s16 + N_A])
            i = j * UNROLL + k
            vals0_v[pl.ds(i * 16, 16)] = vals0_v[pl.ds(i * 16, 16)] + g0
            vals1_v[pl.ds(i * 16, 16)] = vals1_v[pl.ds(i * 16, 16)] + g1
        pltpu.async_copy(vals0_v.at[pl.ds(j * CH, CH)],
                         agg0_sh.at[dst_v.at[j]], sem_s, add=True)
        pltpu.async_copy(vals1_v.at[pl.ds(j * CH, CH)],
                         agg1_sh.at[dst_v.at[j]], sem_s, add=True)
        return carry

    lax.fori_loop(0, NCH, chunk, 0)

    pltpu.make_async_copy(el_hbm.at[0, pl.ds(wid * EPW, EPW)],
                          vals0_v, sem_s).wait()
    pltpu.make_async_copy(el_hbm.at[1, pl.ds(wid * EPW, EPW)],
                          vals1_v, sem_s).wait()
    plsc.subcore_barrier()

    pltpu.sync_copy(agg0_sh.at[pl.ds(sid * RPT, RPT)],
                    out_hbm.at[cid, 0, pl.ds(sid * RPT, RPT)])
    pltpu.sync_copy(agg1_sh.at[pl.ds(sid * RPT, RPT)],
                    out_hbm.at[cid, 1, pl.ds(sid * RPT, RPT)])


@functools.cache
def _sc_agg_kernel():
    return pl.kernel(
        _sc_body,
        out_type=jax.ShapeDtypeStruct((NC, 2, N_A), jnp.float32),
        mesh=plsc.VectorSubcoreMesh(core_axis_name="c", subcore_axis_name="s",
                                    num_cores=NC, num_subcores=NS),
        compiler_params=pltpu.CompilerParams(needs_layout_passes=False),
        scratch_types=[
            pltpu.VMEM((TW,), jnp.float32),
            pltpu.VMEM((NCH, CH), jnp.int32),
            pltpu.VMEM((NCH, CH), jnp.int32),
            pltpu.VMEM((EPW,), jnp.float32),
            pltpu.VMEM((EPW,), jnp.float32),
            pltpu.VMEM_SHARED((N_A,), jnp.float32),
            pltpu.VMEM_SHARED((N_A,), jnp.float32),
            pltpu.SemaphoreType.DMA,
            pltpu.SemaphoreType.DMA,
        ],
    )


def kernel(x, edge_index, edge_attr, W_node, b_node, W_msg, b_msg):
    ei = edge_index.astype(jnp.int32)
    ei_pad = jnp.pad(ei, ((0, 0), (0, E_PAD - E)),
                     constant_values=N).reshape(2, NW, NCH, CH)

    wc = jnp.concatenate([W_node, W_msg[:D]], axis=1)
    bc_t = jnp.concatenate([b_node, jnp.zeros((2,), jnp.float32)])[:, None]
    proj_t, el_t = _proj(wc, bc_t, x, W_msg[D:].T, b_msg[:, None],
                         edge_attr.T)
    proj_flat = jnp.reshape(proj_t, (4 * N,))

    zeros_na = jnp.zeros((N_A,), jnp.float32)

    parts = _sc_agg_kernel()(proj_flat, ei_pad, el_t, zeros_na)
    return _finish(proj_flat, parts).T
